# baseline (speedup 1.0000x reference)
import jax
import jax.numpy as jnp
from jax.experimental import pallas as pl


def _rpn_head_kernel(xp_ref, w9_ref, bc_ref, wh_ref, bh_ref, out_ref):
    m = out_ref.shape[1]
    acc = jnp.broadcast_to(bc_ref[...], (m, bc_ref.shape[1])).astype(jnp.float32)
    for kh in range(3):
        for kw in range(3):
            off = 66 * kh + kw
            s = xp_ref[0, off:off + m, :]
            acc = acc + jnp.dot(s, w9_ref[3 * kh + kw],
                                preferred_element_type=jnp.float32)
    t = jnp.maximum(acc, 0.0)
    out_ref[0] = (jnp.dot(t, wh_ref[...], preferred_element_type=jnp.float32)
                  + bh_ref[...])


def kernel(features, W_conv, b_conv, W_cls, b_cls, W_bbox, b_bbox):
    B, C, H, W = features.shape
    k = W_cls.shape[0]
    k4 = W_bbox.shape[0]
    n_out = k + k4
    Wp = W + 2
    M = H * Wp
    nrows = (H + 2) * Wp
    PADROWS = M + 192

    x = jnp.transpose(features, (0, 2, 3, 1))
    x = jnp.pad(x, ((0, 0), (1, 1), (1, 1), (0, 0)))
    x = x.reshape(B, nrows, C)
    x = jnp.pad(x, ((0, 0), (0, PADROWS - nrows), (0, 0)))

    w9 = jnp.transpose(W_conv, (2, 3, 1, 0)).reshape(9, C, C)
    wh = jnp.concatenate([W_cls[:, :, 0, 0], W_bbox[:, :, 0, 0]], axis=0).T
    bh = jnp.concatenate([b_cls, b_bbox]).reshape(1, n_out)
    bc = b_conv.reshape(1, C)

    out = pl.pallas_call(
        _rpn_head_kernel,
        grid=(B,),
        in_specs=[
            pl.BlockSpec((1, PADROWS, C), lambda b: (b, 0, 0)),
            pl.BlockSpec((9, C, C), lambda b: (0, 0, 0)),
            pl.BlockSpec((1, C), lambda b: (0, 0)),
            pl.BlockSpec((C, n_out), lambda b: (0, 0)),
            pl.BlockSpec((1, n_out), lambda b: (0, 0)),
        ],
        out_specs=pl.BlockSpec((1, M, n_out), lambda b: (b, 0, 0)),
        out_shape=jax.ShapeDtypeStruct((B, M, n_out), jnp.float32),
    )(x, w9, bc, wh, bh)

    y = out.reshape(B, H, Wp, n_out)[:, :, :W, :]
    y = jnp.transpose(y, (0, 3, 1, 2))
    return (y[:, :k], y[:, k:])


# speedup vs baseline: 1.1002x; 1.1002x over previous
import jax
import jax.numpy as jnp
from jax.experimental import pallas as pl


def _rpn_head_kernel(xp_ref, w9_ref, bc_ref, wh_ref, bh_ref, out_ref):
    m = out_ref.shape[1]
    acc = jnp.broadcast_to(bc_ref[...], (m, bc_ref.shape[1])).astype(jnp.float32)
    for kh in range(3):
        for kw in range(3):
            off = 66 * kh + kw
            s = xp_ref[0, off:off + m, :]
            acc = acc + jnp.dot(s, w9_ref[3 * kh + kw],
                                preferred_element_type=jnp.float32)
    t = jnp.maximum(acc, 0.0)
    out_ref[0] = (jnp.dot(t, wh_ref[...], preferred_element_type=jnp.float32)
                  + bh_ref[...])


def kernel(features, W_conv, b_conv, W_cls, b_cls, W_bbox, b_bbox):
    B, C, H, W = features.shape
    k = W_cls.shape[0]
    k4 = W_bbox.shape[0]
    n_out = k + k4
    Wp = W + 2
    M = H * Wp
    nrows = (H + 2) * Wp
    PADROWS = M + 192

    x = jnp.transpose(features, (0, 2, 3, 1))
    x = jnp.pad(x, ((0, 0), (1, 1), (1, 1), (0, 0)))
    x = x.reshape(B, nrows, C)
    x = jnp.pad(x, ((0, 0), (0, PADROWS - nrows), (0, 0)))
    x = x.astype(jnp.bfloat16)

    w9 = jnp.transpose(W_conv, (2, 3, 1, 0)).reshape(9, C, C).astype(jnp.bfloat16)
    wh = jnp.concatenate([W_cls[:, :, 0, 0], W_bbox[:, :, 0, 0]], axis=0).T
    bh = jnp.concatenate([b_cls, b_bbox]).reshape(1, n_out)
    bc = b_conv.reshape(1, C)

    out = pl.pallas_call(
        _rpn_head_kernel,
        grid=(B,),
        in_specs=[
            pl.BlockSpec((1, PADROWS, C), lambda b: (b, 0, 0)),
            pl.BlockSpec((9, C, C), lambda b: (0, 0, 0)),
            pl.BlockSpec((1, C), lambda b: (0, 0)),
            pl.BlockSpec((C, n_out), lambda b: (0, 0)),
            pl.BlockSpec((1, n_out), lambda b: (0, 0)),
        ],
        out_specs=pl.BlockSpec((1, M, n_out), lambda b: (b, 0, 0)),
        out_shape=jax.ShapeDtypeStruct((B, M, n_out), jnp.float32),
    )(x, w9, bc, wh, bh)

    y = out.reshape(B, H, Wp, n_out)[:, :, :W, :]
    y = jnp.transpose(y, (0, 3, 1, 2))
    return (y[:, :k], y[:, k:])


# speedup vs baseline: 1.2938x; 1.1760x over previous
import jax
import jax.numpy as jnp
from jax.experimental import pallas as pl
from jax.experimental.pallas import tpu as pltpu


def _rpn_head_kernel(xf_ref, wall_ref, bc_ref, wh_ref, bh_ref, out_ref,
                     xcol_ref):
    c = xf_ref.shape[1]
    m = out_ref.shape[2]
    for kh in range(3):
        for kw in range(3):
            off = 66 * kh + kw
            xcol_ref[c * (3 * kh + kw):c * (3 * kh + kw + 1), :] = (
                xf_ref[0, :, off:off + m])
    acc = jnp.dot(wall_ref[...], xcol_ref[...],
                  preferred_element_type=jnp.float32)
    t = jnp.maximum(acc + bc_ref[...], 0.0)
    out_ref[0] = (jnp.dot(wh_ref[...], t, preferred_element_type=jnp.float32)
                  + bh_ref[...])


def kernel(features, W_conv, b_conv, W_cls, b_cls, W_bbox, b_bbox):
    B, C, H, W = features.shape
    k = W_cls.shape[0]
    k4 = W_bbox.shape[0]
    n_out = k + k4
    Wp = W + 2
    M = H * Wp
    nflat = (H + 2) * Wp
    LPAD = M + 192

    x = jnp.pad(features, ((0, 0), (0, 0), (1, 1), (1, 1)))
    x = x.reshape(B, C, nflat)
    x = jnp.pad(x, ((0, 0), (0, 0), (0, LPAD - nflat)))
    x = x.astype(jnp.bfloat16)

    wall = jnp.transpose(W_conv, (0, 2, 3, 1)).reshape(C, 9 * C)
    wall = wall.astype(jnp.bfloat16)
    wh = jnp.concatenate([W_cls[:, :, 0, 0], W_bbox[:, :, 0, 0]], axis=0)
    bh = jnp.concatenate([b_cls, b_bbox]).reshape(n_out, 1)
    bc = b_conv.reshape(C, 1)

    out = pl.pallas_call(
        _rpn_head_kernel,
        grid=(B,),
        in_specs=[
            pl.BlockSpec((1, C, LPAD), lambda b: (b, 0, 0)),
            pl.BlockSpec((C, 9 * C), lambda b: (0, 0)),
            pl.BlockSpec((C, 1), lambda b: (0, 0)),
            pl.BlockSpec((n_out, C), lambda b: (0, 0)),
            pl.BlockSpec((n_out, 1), lambda b: (0, 0)),
        ],
        out_specs=pl.BlockSpec((1, n_out, M), lambda b: (b, 0, 0)),
        out_shape=jax.ShapeDtypeStruct((B, n_out, M), jnp.float32),
        scratch_shapes=[pltpu.VMEM((9 * C, M), jnp.bfloat16)],
    )(x, wall, bc, wh, bh)

    y = out.reshape(B, n_out, H, Wp)[:, :, :, :W]
    return (y[:, :k], y[:, k:])


# speedup vs baseline: 2.6722x; 2.0653x over previous
import jax
import jax.numpy as jnp
from jax.experimental import pallas as pl
from jax.experimental.pallas import tpu as pltpu


def _rpn_head_kernel(xf_ref, wall_ref, bc_ref, whc_ref, whb_ref,
                     bhc_ref, bhb_ref, cls_ref, bbox_ref, xcol_ref):
    c = xf_ref.shape[1]
    hw = xf_ref.shape[2]
    x = xf_ref[0].astype(jnp.bfloat16)
    wpos = jax.lax.broadcasted_iota(jnp.int32, (1, hw), 1) % 64
    m_lo = wpos != 0
    m_hi = wpos != 63
    for dy in (-1, 0, 1):
        for dx in (-1, 0, 1):
            d = 64 * dy + dx
            if d > 0:
                s = jnp.concatenate(
                    [x[:, d:], jnp.zeros((c, d), jnp.bfloat16)], axis=1)
            elif d < 0:
                s = jnp.concatenate(
                    [jnp.zeros((c, -d), jnp.bfloat16), x[:, :hw + d]], axis=1)
            else:
                s = x
            if dx == -1:
                s = jnp.where(m_lo, s, jnp.bfloat16(0))
            elif dx == 1:
                s = jnp.where(m_hi, s, jnp.bfloat16(0))
            t_idx = 3 * (dy + 1) + (dx + 1)
            xcol_ref[c * t_idx:c * (t_idx + 1), :] = s
    acc = jnp.dot(wall_ref[...], xcol_ref[...],
                  preferred_element_type=jnp.float32)
    t = jnp.maximum(acc + bc_ref[...], 0.0)
    cls_ref[0] = (jnp.dot(whc_ref[...], t, preferred_element_type=jnp.float32)
                  + bhc_ref[...])
    bbox_ref[0] = (jnp.dot(whb_ref[...], t, preferred_element_type=jnp.float32)
                   + bhb_ref[...])


def kernel(features, W_conv, b_conv, W_cls, b_cls, W_bbox, b_bbox):
    B, C, H, W = features.shape
    k = W_cls.shape[0]
    k4 = W_bbox.shape[0]
    HW = H * W

    xf = features.reshape(B, C, HW)

    wall = jnp.transpose(W_conv, (0, 2, 3, 1)).reshape(C, 9 * C)
    wall = wall.astype(jnp.bfloat16)
    whc = W_cls[:, :, 0, 0]
    whb = W_bbox[:, :, 0, 0]
    bhc = b_cls.reshape(k, 1)
    bhb = b_bbox.reshape(k4, 1)
    bc = b_conv.reshape(C, 1)

    cls_f, bbox_f = pl.pallas_call(
        _rpn_head_kernel,
        grid=(B,),
        in_specs=[
            pl.BlockSpec((1, C, HW), lambda b: (b, 0, 0)),
            pl.BlockSpec((C, 9 * C), lambda b: (0, 0)),
            pl.BlockSpec((C, 1), lambda b: (0, 0)),
            pl.BlockSpec((k, C), lambda b: (0, 0)),
            pl.BlockSpec((k4, C), lambda b: (0, 0)),
            pl.BlockSpec((k, 1), lambda b: (0, 0)),
            pl.BlockSpec((k4, 1), lambda b: (0, 0)),
        ],
        out_specs=[
            pl.BlockSpec((1, k, HW), lambda b: (b, 0, 0)),
            pl.BlockSpec((1, k4, HW), lambda b: (b, 0, 0)),
        ],
        out_shape=[
            jax.ShapeDtypeStruct((B, k, HW), jnp.float32),
            jax.ShapeDtypeStruct((B, k4, HW), jnp.float32),
        ],
        scratch_shapes=[pltpu.VMEM((9 * C, HW), jnp.bfloat16)],
    )(xf, wall, bc, whc, whb, bhc, bhb)

    return (cls_f.reshape(B, k, H, W), bbox_f.reshape(B, k4, H, W))
